# baseline (speedup 1.0000x reference)
import functools

import jax
import jax.numpy as jnp
from jax import lax
from jax.experimental import pallas as pl
from jax.experimental.pallas import tpu as pltpu
from jax.experimental.pallas import tpu_sc as plsc

KTOP = 16
QB = 256
KB = 512
CHUNK = 128

_DOT11 = (((1,), (1,)), ((), ()))


def _proj_body(k_ref, v_ref, wk_ref, bk_ref, wv_ref, bv_ref, kp_ref, vp_ref):
    kb = k_ref[0]
    vb = v_ref[0]
    kp_ref[0] = lax.dot_general(kb, wk_ref[...], _DOT11,
                                preferred_element_type=jnp.float32) + bk_ref[...]
    vp_ref[0] = lax.dot_general(vb, wv_ref[...], _DOT11,
                                preferred_element_type=jnp.float32) + bv_ref[...]


def _select_body(q_ref, k_ref, wq_ref, bq_ref, qe_ref, idx_ref, *, nk, boff):
    qb = q_ref[...]
    kb = k_ref[...]
    qe_ref[...] = lax.dot_general(qb, wq_ref[...], _DOT11,
                                  preferred_element_type=jnp.float32) + bq_ref[...]
    q2 = jnp.sum(qb * qb, axis=1)
    k2 = jnp.sum(kb * kb, axis=1)
    qk = lax.dot_general(qb, kb, _DOT11, preferred_element_type=jnp.float32)
    s = q2[:, None] + k2[None, :] - 2.0 * qk
    fiota = lax.broadcasted_iota(jnp.int32, (QB, nk), 1).astype(jnp.float32)
    big = jnp.float32(3.0e38)
    cols = []
    for _ in range(KTOP):
        m = jnp.min(s, axis=1, keepdims=True)
        fidx = jnp.min(jnp.where(s == m, fiota, big), axis=1)
        cols.append(fidx)
        s = jnp.where(fiota == fidx[:, None], big, s)
    idx_ref[...] = jnp.stack(cols, axis=1).astype(jnp.int32) + boff


def _attn_body(qe_ref, kg_ref, vg_ref, o_ref):
    qe = qe_ref[...]
    s = jnp.sum(qe[:, None, :] * kg_ref[...], axis=2)
    m = jnp.max(s, axis=1, keepdims=True)
    e = jnp.exp(s - m)
    a = e / jnp.sum(e, axis=1, keepdims=True)
    o_ref[...] = jnp.sum(a[:, :, None] * vg_ref[...], axis=1)


def _gather_pairs(kp2, vp2, idxf):
    n, h = idxf.shape[0], kp2.shape[1]
    info = plsc.get_sparse_core_info()
    nw = info.num_cores * info.num_subcores
    per_w = n // nw
    nchunk = per_w // CHUNK
    mesh = plsc.VectorSubcoreMesh(core_axis_name="c", subcore_axis_name="s")

    @functools.partial(
        pl.kernel, mesh=mesh,
        out_type=(jax.ShapeDtypeStruct((n, h), jnp.float32),
                  jax.ShapeDtypeStruct((n, h), jnp.float32)),
        scratch_types=[pltpu.VMEM((CHUNK,), jnp.int32),
                       pltpu.VMEM((CHUNK, h), jnp.float32),
                       pltpu.VMEM((CHUNK, h), jnp.float32),
                       pltpu.SemaphoreType.DMA,
                       pltpu.SemaphoreType.DMA],
    )
    def body(kp_hbm, vp_hbm, idx_hbm, kg_hbm, vg_hbm, idx_v, kbuf, vbuf, sk, sv):
        wid = lax.axis_index("s") * info.num_cores + lax.axis_index("c")
        base = wid * per_w

        def step(c, carry):
            off = base + c * CHUNK
            pltpu.sync_copy(idx_hbm.at[pl.ds(off, CHUNK)], idx_v)
            ck = pltpu.async_copy(kp_hbm.at[idx_v], kbuf, sk)
            cv = pltpu.async_copy(vp_hbm.at[idx_v], vbuf, sv)
            ck.wait()
            pltpu.sync_copy(kbuf, kg_hbm.at[pl.ds(off, CHUNK)])
            cv.wait()
            pltpu.sync_copy(vbuf, vg_hbm.at[pl.ds(off, CHUNK)])
            return carry

        lax.fori_loop(0, nchunk, step, 0)

    return body(kp2, vp2, idxf)


def kernel(q, k, v, Wq, bq, Wk, bk, Wv, bv, top_k):
    b, nq, c = q.shape
    nk = k.shape[1]
    h = Wq.shape[0]
    bq2, bk2, bv2 = (x.reshape(1, h) for x in (bq, bk, bv))
    f32 = jnp.float32

    w2_spec = pl.BlockSpec((h, c), lambda i, j: (0, 0))
    b2_spec = pl.BlockSpec((1, h), lambda i, j: (0, 0))
    w_spec = pl.BlockSpec((h, c), lambda i: (0, 0))
    b_spec = pl.BlockSpec((1, h), lambda i: (0, 0))

    kp, vp = pl.pallas_call(
        _proj_body,
        grid=(b, nk // KB),
        in_specs=[pl.BlockSpec((1, KB, c), lambda i, j: (i, j, 0)),
                  pl.BlockSpec((1, KB, c), lambda i, j: (i, j, 0)),
                  w2_spec, b2_spec, w2_spec, b2_spec],
        out_specs=[pl.BlockSpec((1, KB, h), lambda i, j: (i, j, 0)),
                   pl.BlockSpec((1, KB, h), lambda i, j: (i, j, 0))],
        out_shape=[jax.ShapeDtypeStruct((b, nk, h), f32),
                   jax.ShapeDtypeStruct((b, nk, h), f32)],
    )(k, v, Wk, bk2, Wv, bv2)
    kp2 = kp.reshape(b * nk, h)
    vp2 = vp.reshape(b * nk, h)

    qes, idxs = [], []
    for bi in range(b):
        qe, idx = pl.pallas_call(
            functools.partial(_select_body, nk=nk, boff=bi * nk),
            grid=(nq // QB,),
            in_specs=[pl.BlockSpec((QB, c), lambda i: (i, 0)),
                      pl.BlockSpec((nk, c), lambda i: (0, 0)),
                      w_spec, b_spec],
            out_specs=[pl.BlockSpec((QB, h), lambda i: (i, 0)),
                       pl.BlockSpec((QB, KTOP), lambda i: (i, 0))],
            out_shape=[jax.ShapeDtypeStruct((nq, h), f32),
                       jax.ShapeDtypeStruct((nq, KTOP), jnp.int32)],
        )(q[bi], k[bi], Wq, bq2)
        qes.append(qe)
        idxs.append(idx)

    gathered = [_gather_pairs(kp2, vp2, idxs[bi].reshape(-1))
                for bi in range(b)]

    outs = []
    for bi in range(b):
        kg, vg = gathered[bi]
        out_b = pl.pallas_call(
            _attn_body,
            grid=(nq // QB,),
            in_specs=[pl.BlockSpec((QB, h), lambda i: (i, 0)),
                      pl.BlockSpec((QB, KTOP, h), lambda i: (i, 0, 0)),
                      pl.BlockSpec((QB, KTOP, h), lambda i: (i, 0, 0))],
            out_specs=pl.BlockSpec((QB, h), lambda i: (i, 0)),
            out_shape=jax.ShapeDtypeStruct((nq, h), f32),
        )(qes[bi], kg.reshape(nq, KTOP, h), vg.reshape(nq, KTOP, h))
        outs.append(out_b)

    return jnp.stack(outs, axis=0)


# speedup vs baseline: 1.1178x; 1.1178x over previous
import functools

import jax
import jax.numpy as jnp
from jax import lax
from jax.experimental import pallas as pl
from jax.experimental.pallas import tpu as pltpu
from jax.experimental.pallas import tpu_sc as plsc

KTOP = 16
QB = 256
KB = 512
CHUNK = 128

_DOT11 = (((1,), (1,)), ((), ()))


def _proj_body(k_ref, v_ref, wk_ref, bk_ref, wv_ref, bv_ref, kp_ref, vp_ref):
    kb = k_ref[0]
    vb = v_ref[0]
    kp_ref[0] = lax.dot_general(kb, wk_ref[...], _DOT11,
                                preferred_element_type=jnp.float32) + bk_ref[...]
    vp_ref[0] = lax.dot_general(vb, wv_ref[...], _DOT11,
                                preferred_element_type=jnp.float32) + bv_ref[...]


def _select_body(q_ref, k_ref, wq_ref, bq_ref, qe_ref, idx_ref, *, nk, boff):
    qb = q_ref[...]
    kb = k_ref[...]
    qe_ref[...] = lax.dot_general(qb, wq_ref[...], _DOT11,
                                  preferred_element_type=jnp.float32) + bq_ref[...]
    q2 = jnp.sum(qb * qb, axis=1)
    k2 = jnp.sum(kb * kb, axis=1)
    qk = lax.dot_general(qb, kb, _DOT11, preferred_element_type=jnp.float32)
    s = q2[:, None] + k2[None, :] - 2.0 * qk
    fiota = lax.broadcasted_iota(jnp.int32, (QB, nk), 1).astype(jnp.float32)
    big = jnp.float32(3.0e38)
    cols = []
    for _ in range(KTOP):
        m = jnp.min(s, axis=1, keepdims=True)
        fidx = jnp.min(jnp.where(s == m, fiota, big), axis=1)
        cols.append(fidx)
        s = jnp.where(fiota == fidx[:, None], big, s)
    idx_ref[...] = jnp.stack(cols, axis=1).astype(jnp.int32) + boff


def _attn_body(qe_ref, kg_ref, vg_ref, o_ref):
    qe = qe_ref[...]
    s = jnp.sum(qe[:, None, :] * kg_ref[...], axis=2)
    m = jnp.max(s, axis=1, keepdims=True)
    e = jnp.exp(s - m)
    a = e / jnp.sum(e, axis=1, keepdims=True)
    o_ref[...] = jnp.sum(a[:, :, None] * vg_ref[...], axis=1)


QC = 8

_GDN = lax.GatherDimensionNumbers(offset_dims=(), collapsed_slice_dims=(0,),
                                  start_index_map=(0,))


def _vtake(x, idx):
    return lax.gather(x, idx[:, None], _GDN, (1,),
                      mode=lax.GatherScatterMode.PROMISE_IN_BOUNDS)


def _sc_attend(kp2, vp2, idxf, qe):
    n, h = idxf.shape[0], kp2.shape[1]
    nq = qe.shape[0]
    info = plsc.get_sparse_core_info()
    nw = info.num_cores * info.num_subcores
    q_per_w = nq // nw
    nchunk = q_per_w // QC
    hc = h // 16
    mesh = plsc.VectorSubcoreMesh(core_axis_name="c", subcore_axis_name="s")

    @functools.partial(
        pl.kernel, mesh=mesh,
        out_type=jax.ShapeDtypeStruct((nq, h), jnp.float32),
        scratch_types=[pltpu.VMEM((CHUNK,), jnp.int32),
                       pltpu.VMEM((CHUNK, h), jnp.float32),
                       pltpu.VMEM((CHUNK, h), jnp.float32),
                       pltpu.VMEM((q_per_w, h), jnp.float32),
                       pltpu.VMEM((QC, h), jnp.float32),
                       pltpu.SemaphoreType.DMA,
                       pltpu.SemaphoreType.DMA],
    )
    def body(kp_hbm, vp_hbm, idx_hbm, qe_hbm, o_hbm,
             idx_v, kbuf, vbuf, qbuf, obuf, sk, sv):
        wid = lax.axis_index("s") * info.num_cores + lax.axis_index("c")
        qbase = wid * q_per_w
        pltpu.sync_copy(qe_hbm.at[pl.ds(qbase, q_per_w)], qbuf)
        li = lax.iota(jnp.int32, 16)

        def query(qi, c):
            qrow = c * QC + qi
            q_vecs = [qbuf[qrow, pl.ds(j * 16, 16)] for j in range(hc)]
            s_vec = jnp.zeros((16,), jnp.float32)
            for i in range(KTOP):
                r = qi * KTOP + i
                acc = kbuf[r, pl.ds(0, 16)] * q_vecs[0]
                for j in range(1, hc):
                    acc = acc + kbuf[r, pl.ds(j * 16, 16)] * q_vecs[j]
                for m in (8, 4, 2, 1):
                    acc = acc + _vtake(acc, jnp.bitwise_xor(li, m))
                s_vec = jnp.where(li == i, acc, s_vec)
            mx = s_vec
            for m in (8, 4, 2, 1):
                mx = jnp.maximum(mx, _vtake(mx, jnp.bitwise_xor(li, m)))
            e = jnp.exp(s_vec - mx)
            den = e
            for m in (8, 4, 2, 1):
                den = den + _vtake(den, jnp.bitwise_xor(li, m))
            a = e / den
            a_b = [_vtake(a, jnp.full((16,), i, jnp.int32)) for i in range(KTOP)]
            for j in range(hc):
                o = a_b[0] * vbuf[qi * KTOP, pl.ds(j * 16, 16)]
                for i in range(1, KTOP):
                    o = o + a_b[i] * vbuf[qi * KTOP + i, pl.ds(j * 16, 16)]
                obuf[qi, pl.ds(j * 16, 16)] = o
            return c

        def step(c, carry):
            off = (qbase + c * QC) * KTOP
            pltpu.sync_copy(idx_hbm.at[pl.ds(off, CHUNK)], idx_v)
            ck = pltpu.async_copy(kp_hbm.at[idx_v], kbuf, sk)
            cv = pltpu.async_copy(vp_hbm.at[idx_v], vbuf, sv)
            ck.wait()
            cv.wait()
            lax.fori_loop(0, QC, query, c)
            pltpu.sync_copy(obuf, o_hbm.at[pl.ds(qbase + c * QC, QC)])
            return carry

        lax.fori_loop(0, nchunk, step, 0)

    return body(kp2, vp2, idxf, qe)


def kernel(q, k, v, Wq, bq, Wk, bk, Wv, bv, top_k):
    b, nq, c = q.shape
    nk = k.shape[1]
    h = Wq.shape[0]
    bq2, bk2, bv2 = (x.reshape(1, h) for x in (bq, bk, bv))
    f32 = jnp.float32

    w2_spec = pl.BlockSpec((h, c), lambda i, j: (0, 0))
    b2_spec = pl.BlockSpec((1, h), lambda i, j: (0, 0))
    w_spec = pl.BlockSpec((h, c), lambda i: (0, 0))
    b_spec = pl.BlockSpec((1, h), lambda i: (0, 0))

    kp, vp = pl.pallas_call(
        _proj_body,
        grid=(b, nk // KB),
        in_specs=[pl.BlockSpec((1, KB, c), lambda i, j: (i, j, 0)),
                  pl.BlockSpec((1, KB, c), lambda i, j: (i, j, 0)),
                  w2_spec, b2_spec, w2_spec, b2_spec],
        out_specs=[pl.BlockSpec((1, KB, h), lambda i, j: (i, j, 0)),
                   pl.BlockSpec((1, KB, h), lambda i, j: (i, j, 0))],
        out_shape=[jax.ShapeDtypeStruct((b, nk, h), f32),
                   jax.ShapeDtypeStruct((b, nk, h), f32)],
    )(k, v, Wk, bk2, Wv, bv2)
    kp2 = kp.reshape(b * nk, h)
    vp2 = vp.reshape(b * nk, h)

    qes, idxs = [], []
    for bi in range(b):
        qe, idx = pl.pallas_call(
            functools.partial(_select_body, nk=nk, boff=bi * nk),
            grid=(nq // QB,),
            in_specs=[pl.BlockSpec((QB, c), lambda i: (i, 0)),
                      pl.BlockSpec((nk, c), lambda i: (0, 0)),
                      w_spec, b_spec],
            out_specs=[pl.BlockSpec((QB, h), lambda i: (i, 0)),
                       pl.BlockSpec((QB, KTOP), lambda i: (i, 0))],
            out_shape=[jax.ShapeDtypeStruct((nq, h), f32),
                       jax.ShapeDtypeStruct((nq, KTOP), jnp.int32)],
        )(q[bi], k[bi], Wq, bq2)
        qes.append(qe)
        idxs.append(idx)

    outs = [_sc_attend(kp2, vp2, idxs[bi].reshape(-1), qes[bi])
            for bi in range(b)]

    return jnp.stack(outs, axis=0)


# speedup vs baseline: 1.1388x; 1.0187x over previous
import functools

import jax
import jax.numpy as jnp
from jax import lax
from jax.experimental import pallas as pl
from jax.experimental.pallas import tpu as pltpu
from jax.experimental.pallas import tpu_sc as plsc

KTOP = 16
QB = 256
KB = 512
CHUNK = 128

_DOT11 = (((1,), (1,)), ((), ()))


def _proj_body(k_ref, v_ref, wk_ref, bk_ref, wv_ref, bv_ref, kp_ref, vp_ref):
    kb = k_ref[0]
    vb = v_ref[0]
    kp_ref[0] = lax.dot_general(kb, wk_ref[...], _DOT11,
                                preferred_element_type=jnp.float32) + bk_ref[...]
    vp_ref[0] = lax.dot_general(vb, wv_ref[...], _DOT11,
                                preferred_element_type=jnp.float32) + bv_ref[...]


def _select_body(q_ref, k_ref, wq_ref, bq_ref, qe_ref, idx_ref, *, nk, boff):
    qb = q_ref[...]
    kb = k_ref[...]
    qe_ref[...] = lax.dot_general(qb, wq_ref[...], _DOT11,
                                  preferred_element_type=jnp.float32) + bq_ref[...]
    q2 = jnp.sum(qb * qb, axis=1)
    k2 = jnp.sum(kb * kb, axis=1)
    qk = lax.dot_general(qb, kb, _DOT11, preferred_element_type=jnp.float32)
    s = q2[:, None] + k2[None, :] - 2.0 * qk
    fiota = lax.broadcasted_iota(jnp.int32, (QB, nk), 1).astype(jnp.float32)
    big = jnp.float32(3.0e38)
    cols = []
    for _ in range(KTOP):
        m = jnp.min(s, axis=1, keepdims=True)
        fidx = jnp.min(jnp.where(s == m, fiota, big), axis=1)
        cols.append(fidx)
        s = jnp.where(fiota == fidx[:, None], big, s)
    idx_ref[...] = jnp.stack(cols, axis=1).astype(jnp.int32) + boff


def _attn_body(qe_ref, kg_ref, vg_ref, o_ref):
    qe = qe_ref[...]
    s = jnp.sum(qe[:, None, :] * kg_ref[...], axis=2)
    m = jnp.max(s, axis=1, keepdims=True)
    e = jnp.exp(s - m)
    a = e / jnp.sum(e, axis=1, keepdims=True)
    o_ref[...] = jnp.sum(a[:, :, None] * vg_ref[...], axis=1)


QC = 8

_GDN = lax.GatherDimensionNumbers(offset_dims=(), collapsed_slice_dims=(0,),
                                  start_index_map=(0,))


def _vtake(x, idx):
    return lax.gather(x, idx[:, None], _GDN, (1,),
                      mode=lax.GatherScatterMode.PROMISE_IN_BOUNDS)


def _sc_attend(kp2, vp2, idxf, qe):
    n, h = idxf.shape[0], kp2.shape[1]
    nq = qe.shape[0]
    info = plsc.get_sparse_core_info()
    nw = info.num_cores * info.num_subcores
    q_per_w = nq // nw
    nchunk = q_per_w // QC
    hc = h // 16
    mesh = plsc.VectorSubcoreMesh(core_axis_name="c", subcore_axis_name="s")

    @functools.partial(
        pl.kernel, mesh=mesh,
        out_type=jax.ShapeDtypeStruct((nq, h), jnp.float32),
        scratch_types=[pltpu.VMEM((CHUNK,), jnp.int32),
                       pltpu.VMEM((CHUNK,), jnp.int32),
                       pltpu.VMEM((CHUNK, h), jnp.float32),
                       pltpu.VMEM((CHUNK, h), jnp.float32),
                       pltpu.VMEM((CHUNK, h), jnp.float32),
                       pltpu.VMEM((CHUNK, h), jnp.float32),
                       pltpu.VMEM((q_per_w, h), jnp.float32),
                       pltpu.VMEM((QC, h), jnp.float32),
                       pltpu.SemaphoreType.DMA,
                       pltpu.SemaphoreType.DMA,
                       pltpu.SemaphoreType.DMA,
                       pltpu.SemaphoreType.DMA],
    )
    def body(kp_hbm, vp_hbm, idx_hbm, qe_hbm, o_hbm,
             idx0, idx1, kbuf0, kbuf1, vbuf0, vbuf1, qbuf, obuf,
             sk0, sk1, sv0, sv1):
        wid = lax.axis_index("s") * info.num_cores + lax.axis_index("c")
        qbase = wid * q_per_w
        pltpu.sync_copy(qe_hbm.at[pl.ds(qbase, q_per_w)], qbuf)
        li = lax.iota(jnp.int32, 16)
        bufs = ((idx0, kbuf0, vbuf0, sk0, sv0), (idx1, kbuf1, vbuf1, sk1, sv1))

        def fire(c, p):
            iv, kb, vb, sk, sv = bufs[p]
            pltpu.sync_copy(idx_hbm.at[pl.ds((qbase + c * QC) * KTOP, CHUNK)], iv)
            pltpu.async_copy(kp_hbm.at[iv], kb, sk)
            pltpu.async_copy(vp_hbm.at[iv], vb, sv)

        def wait(p):
            iv, kb, vb, sk, sv = bufs[p]
            pltpu.make_async_copy(kp_hbm.at[iv], kb, sk).wait()
            pltpu.make_async_copy(vp_hbm.at[iv], vb, sv).wait()

        def make_query(kb, vb):
            def query(qi, c):
                qrow = c * QC + qi
                q_vecs = [qbuf[qrow, pl.ds(j * 16, 16)] for j in range(hc)]
                s_vec = jnp.zeros((16,), jnp.float32)
                for i in range(KTOP):
                    r = qi * KTOP + i
                    acc = kb[r, pl.ds(0, 16)] * q_vecs[0]
                    for j in range(1, hc):
                        acc = acc + kb[r, pl.ds(j * 16, 16)] * q_vecs[j]
                    for m in (8, 4, 2, 1):
                        acc = acc + _vtake(acc, jnp.bitwise_xor(li, m))
                    s_vec = jnp.where(li == i, acc, s_vec)
                mx = s_vec
                for m in (8, 4, 2, 1):
                    mx = jnp.maximum(mx, _vtake(mx, jnp.bitwise_xor(li, m)))
                e = jnp.exp(s_vec - mx)
                den = e
                for m in (8, 4, 2, 1):
                    den = den + _vtake(den, jnp.bitwise_xor(li, m))
                a = e / den
                a_b = [_vtake(a, jnp.full((16,), i, jnp.int32))
                       for i in range(KTOP)]
                for j in range(hc):
                    o = a_b[0] * vb[qi * KTOP, pl.ds(j * 16, 16)]
                    for i in range(1, KTOP):
                        o = o + a_b[i] * vb[qi * KTOP + i, pl.ds(j * 16, 16)]
                    obuf[qi, pl.ds(j * 16, 16)] = o
                return c
            return query

        def compute(c, p):
            _, kb, vb, _, _ = bufs[p]
            lax.fori_loop(0, QC, make_query(kb, vb), c)
            pltpu.sync_copy(obuf, o_hbm.at[pl.ds(qbase + c * QC, QC)])

        fire(0, 0)

        def pair(t, carry):
            c0 = 2 * t
            fire(c0 + 1, 1)
            wait(0)
            compute(c0, 0)

            @pl.when(c0 + 2 < nchunk)
            def _():
                fire(c0 + 2, 0)

            wait(1)
            compute(c0 + 1, 1)
            return carry

        lax.fori_loop(0, nchunk // 2, pair, 0)

    return body(kp2, vp2, idxf, qe)


def kernel(q, k, v, Wq, bq, Wk, bk, Wv, bv, top_k):
    b, nq, c = q.shape
    nk = k.shape[1]
    h = Wq.shape[0]
    bq2, bk2, bv2 = (x.reshape(1, h) for x in (bq, bk, bv))
    f32 = jnp.float32

    w2_spec = pl.BlockSpec((h, c), lambda i, j: (0, 0))
    b2_spec = pl.BlockSpec((1, h), lambda i, j: (0, 0))
    w_spec = pl.BlockSpec((h, c), lambda i: (0, 0))
    b_spec = pl.BlockSpec((1, h), lambda i: (0, 0))

    kp, vp = pl.pallas_call(
        _proj_body,
        grid=(b, nk // KB),
        in_specs=[pl.BlockSpec((1, KB, c), lambda i, j: (i, j, 0)),
                  pl.BlockSpec((1, KB, c), lambda i, j: (i, j, 0)),
                  w2_spec, b2_spec, w2_spec, b2_spec],
        out_specs=[pl.BlockSpec((1, KB, h), lambda i, j: (i, j, 0)),
                   pl.BlockSpec((1, KB, h), lambda i, j: (i, j, 0))],
        out_shape=[jax.ShapeDtypeStruct((b, nk, h), f32),
                   jax.ShapeDtypeStruct((b, nk, h), f32)],
    )(k, v, Wk, bk2, Wv, bv2)
    kp2 = kp.reshape(b * nk, h)
    vp2 = vp.reshape(b * nk, h)

    qes, idxs = [], []
    for bi in range(b):
        qe, idx = pl.pallas_call(
            functools.partial(_select_body, nk=nk, boff=bi * nk),
            grid=(nq // QB,),
            in_specs=[pl.BlockSpec((QB, c), lambda i: (i, 0)),
                      pl.BlockSpec((nk, c), lambda i: (0, 0)),
                      w_spec, b_spec],
            out_specs=[pl.BlockSpec((QB, h), lambda i: (i, 0)),
                       pl.BlockSpec((QB, KTOP), lambda i: (i, 0))],
            out_shape=[jax.ShapeDtypeStruct((nq, h), f32),
                       jax.ShapeDtypeStruct((nq, KTOP), jnp.int32)],
        )(q[bi], k[bi], Wq, bq2)
        qes.append(qe)
        idxs.append(idx)

    outs = [_sc_attend(kp2, vp2, idxs[bi].reshape(-1), qes[bi])
            for bi in range(b)]

    return jnp.stack(outs, axis=0)


# speedup vs baseline: 1.1391x; 1.0002x over previous
import functools

import jax
import jax.numpy as jnp
from jax import lax
from jax.experimental import pallas as pl
from jax.experimental.pallas import tpu as pltpu
from jax.experimental.pallas import tpu_sc as plsc

KTOP = 16
QB = 256
KB = 512
CHUNK = 128

_DOT11 = (((1,), (1,)), ((), ()))


def _proj_body(k_ref, v_ref, wk_ref, bk_ref, wv_ref, bv_ref, kp_ref, vp_ref):
    kb = k_ref[0]
    vb = v_ref[0]
    kp_ref[0] = lax.dot_general(kb, wk_ref[...], _DOT11,
                                preferred_element_type=jnp.float32) + bk_ref[...]
    vp_ref[0] = lax.dot_general(vb, wv_ref[...], _DOT11,
                                preferred_element_type=jnp.float32) + bv_ref[...]


def _select_body(q_ref, k_ref, wq_ref, bq_ref, qe_ref, idx_ref, *, nk, boff):
    qb = q_ref[...]
    kb = k_ref[...]
    qe_ref[...] = lax.dot_general(qb, wq_ref[...], _DOT11,
                                  preferred_element_type=jnp.float32) + bq_ref[...]
    q2 = jnp.sum(qb * qb, axis=1)
    k2 = jnp.sum(kb * kb, axis=1)
    qk = lax.dot_general(qb, kb, _DOT11, preferred_element_type=jnp.float32)
    s = q2[:, None] + k2[None, :] - 2.0 * qk
    fiota = lax.broadcasted_iota(jnp.int32, (QB, nk), 1).astype(jnp.float32)
    big = jnp.float32(3.0e38)
    cols = []
    for _ in range(KTOP):
        m = jnp.min(s, axis=1, keepdims=True)
        fidx = jnp.min(jnp.where(s == m, fiota, big), axis=1)
        cols.append(fidx)
        s = jnp.where(fiota == fidx[:, None], big, s)
    idx_ref[...] = jnp.stack(cols, axis=1).astype(jnp.int32) + boff


QC = 8

_GDN = lax.GatherDimensionNumbers(offset_dims=(), collapsed_slice_dims=(0,),
                                  start_index_map=(0,))


def _vtake(x, idx):
    return lax.gather(x, idx[:, None], _GDN, (1,),
                      mode=lax.GatherScatterMode.PROMISE_IN_BOUNDS)


def _sc_attend(kp2, vp2, idxf, qe):
    n, h = idxf.shape[0], kp2.shape[1]
    nq = qe.shape[0]
    info = plsc.get_sparse_core_info()
    nw = info.num_cores * info.num_subcores
    q_per_w = nq // nw
    nchunk = q_per_w // QC
    hc = h // 16
    mesh = plsc.VectorSubcoreMesh(core_axis_name="c", subcore_axis_name="s")

    @functools.partial(
        pl.kernel, mesh=mesh,
        out_type=jax.ShapeDtypeStruct((nq, h), jnp.float32),
        scratch_types=[pltpu.VMEM((CHUNK,), jnp.int32),
                       pltpu.VMEM((CHUNK,), jnp.int32),
                       pltpu.VMEM((CHUNK, h), jnp.float32),
                       pltpu.VMEM((CHUNK, h), jnp.float32),
                       pltpu.VMEM((CHUNK, h), jnp.float32),
                       pltpu.VMEM((CHUNK, h), jnp.float32),
                       pltpu.VMEM((q_per_w, h), jnp.float32),
                       pltpu.VMEM((QC, h), jnp.float32),
                       pltpu.SemaphoreType.DMA,
                       pltpu.SemaphoreType.DMA,
                       pltpu.SemaphoreType.DMA,
                       pltpu.SemaphoreType.DMA],
    )
    def body(kp_hbm, vp_hbm, idx_hbm, qe_hbm, o_hbm,
             idx0, idx1, kbuf0, kbuf1, vbuf0, vbuf1, qbuf, obuf,
             sk0, sk1, sv0, sv1):
        wid = lax.axis_index("s") * info.num_cores + lax.axis_index("c")
        qbase = wid * q_per_w
        pltpu.sync_copy(qe_hbm.at[pl.ds(qbase, q_per_w)], qbuf)
        li = lax.iota(jnp.int32, 16)
        bufs = ((idx0, kbuf0, vbuf0, sk0, sv0), (idx1, kbuf1, vbuf1, sk1, sv1))

        def fire(c, p):
            iv, kb, vb, sk, sv = bufs[p]
            pltpu.sync_copy(idx_hbm.at[pl.ds((qbase + c * QC) * KTOP, CHUNK)], iv)
            pltpu.async_copy(kp_hbm.at[iv], kb, sk)
            pltpu.async_copy(vp_hbm.at[iv], vb, sv)

        def wait(p):
            iv, kb, vb, sk, sv = bufs[p]
            pltpu.make_async_copy(kp_hbm.at[iv], kb, sk).wait()
            pltpu.make_async_copy(vp_hbm.at[iv], vb, sv).wait()

        def make_query(kb, vb):
            def query(qi, c):
                qrow = c * QC + qi
                q_vecs = [qbuf[qrow, pl.ds(j * 16, 16)] for j in range(hc)]
                s_vec = jnp.zeros((16,), jnp.float32)
                for i in range(KTOP):
                    r = qi * KTOP + i
                    acc = kb[r, pl.ds(0, 16)] * q_vecs[0]
                    for j in range(1, hc):
                        acc = acc + kb[r, pl.ds(j * 16, 16)] * q_vecs[j]
                    for m in (8, 4, 2, 1):
                        acc = acc + _vtake(acc, jnp.bitwise_xor(li, m))
                    s_vec = jnp.where(li == i, acc, s_vec)
                mx = s_vec
                for m in (8, 4, 2, 1):
                    mx = jnp.maximum(mx, _vtake(mx, jnp.bitwise_xor(li, m)))
                e = jnp.exp(s_vec - mx)
                den = e
                for m in (8, 4, 2, 1):
                    den = den + _vtake(den, jnp.bitwise_xor(li, m))
                a = e / den
                a_b = [_vtake(a, jnp.full((16,), i, jnp.int32))
                       for i in range(KTOP)]
                for j in range(hc):
                    o = a_b[0] * vb[qi * KTOP, pl.ds(j * 16, 16)]
                    for i in range(1, KTOP):
                        o = o + a_b[i] * vb[qi * KTOP + i, pl.ds(j * 16, 16)]
                    obuf[qi, pl.ds(j * 16, 16)] = o
                return c
            return query

        def compute(c, p):
            _, kb, vb, _, _ = bufs[p]
            lax.fori_loop(0, QC, make_query(kb, vb), c)
            pltpu.sync_copy(obuf, o_hbm.at[pl.ds(qbase + c * QC, QC)])

        fire(0, 0)

        def pair(t, carry):
            c0 = 2 * t
            fire(c0 + 1, 1)
            wait(0)
            compute(c0, 0)

            @pl.when(c0 + 2 < nchunk)
            def _():
                fire(c0 + 2, 0)

            wait(1)
            compute(c0 + 1, 1)
            return carry

        lax.fori_loop(0, nchunk // 2, pair, 0)

    return body(kp2, vp2, idxf, qe)


def kernel(q, k, v, Wq, bq, Wk, bk, Wv, bv, top_k):
    b, nq, c = q.shape
    nk = k.shape[1]
    h = Wq.shape[0]
    bq2, bk2, bv2 = (x.reshape(1, h) for x in (bq, bk, bv))
    f32 = jnp.float32

    w2_spec = pl.BlockSpec((h, c), lambda i, j: (0, 0))
    b2_spec = pl.BlockSpec((1, h), lambda i, j: (0, 0))
    w_spec = pl.BlockSpec((h, c), lambda i: (0, 0))
    b_spec = pl.BlockSpec((1, h), lambda i: (0, 0))

    kp, vp = pl.pallas_call(
        _proj_body,
        grid=(b, nk // KB),
        in_specs=[pl.BlockSpec((1, KB, c), lambda i, j: (i, j, 0)),
                  pl.BlockSpec((1, KB, c), lambda i, j: (i, j, 0)),
                  w2_spec, b2_spec, w2_spec, b2_spec],
        out_specs=[pl.BlockSpec((1, KB, h), lambda i, j: (i, j, 0)),
                   pl.BlockSpec((1, KB, h), lambda i, j: (i, j, 0))],
        out_shape=[jax.ShapeDtypeStruct((b, nk, h), f32),
                   jax.ShapeDtypeStruct((b, nk, h), f32)],
    )(k, v, Wk, bk2, Wv, bv2)
    kp2 = kp.reshape(b * nk, h)
    vp2 = vp.reshape(b * nk, h)

    qes, idxs = [], []
    for bi in range(b):
        qe, idx = pl.pallas_call(
            functools.partial(_select_body, nk=nk, boff=bi * nk),
            grid=(nq // QB,),
            in_specs=[pl.BlockSpec((QB, c), lambda i: (i, 0)),
                      pl.BlockSpec((nk, c), lambda i: (0, 0)),
                      w_spec, b_spec],
            out_specs=[pl.BlockSpec((QB, h), lambda i: (i, 0)),
                       pl.BlockSpec((QB, KTOP), lambda i: (i, 0))],
            out_shape=[jax.ShapeDtypeStruct((nq, h), f32),
                       jax.ShapeDtypeStruct((nq, KTOP), jnp.int32)],
        )(q[bi], k[bi], Wq, bq2)
        qes.append(qe)
        idxs.append(idx)

    outs = [_sc_attend(kp2, vp2, idxs[bi].reshape(-1), qes[bi])
            for bi in range(b)]

    return jnp.stack(outs, axis=0)
